# Initial kernel scaffold; baseline (speedup 1.0000x reference)
import jax
import jax.numpy as jnp
from jax.experimental import pallas as pl


def kernel(eps, b, points, p):
    raise NotImplementedError("write your pallas kernel here")


# speedup vs baseline: 181.2581x; 181.2581x over previous
import functools

import jax
import jax.numpy as jnp
from jax import lax
from jax.experimental import pallas as pl
from jax.experimental.pallas import tpu as pltpu
from jax.experimental.pallas import tpu_sc as plsc

VMIN = -5.0
VMAX = 5.0
N_KNOTS = 100
INT_LENGTH = (VMAX - VMIN) / (N_KNOTS - 1)
INV_H = 1.0 / INT_LENGTH
BATCH = 1000000

NC, NS, L = 2, 16, 16
NW = NC * NS
CHUNK = 31264
NPAD = NW * CHUNK
TPAD = 112


def _pwl_body(eps_hbm, b_hbm, pts_hbm, p_hbm, out_hbm,
              eps_v, out_v, pts_v, w_v, db_v, tmp_v, pp_v, b_v, sem):
    wid = lax.axis_index("s") * NC + lax.axis_index("c")

    pltpu.sync_copy(pts_hbm, pts_v)
    pltpu.sync_copy(p_hbm, pp_v)
    pltpu.sync_copy(b_hbm, b_v)
    b_s = b_v[...][0]

    base = wid * CHUNK
    cp = pltpu.async_copy(eps_hbm.at[pl.ds(base, CHUNK)], eps_v, sem)

    for c in range(TPAD // L):
        w_v[pl.ds(c * L, L)] = jnp.exp(pp_v[pl.ds(c * L, L)]) + 0.001

    lane = lax.iota(jnp.int32, L)
    for c in range(TPAD // L):
        v = INT_LENGTH * w_v[pl.ds(c * L, L)]
        if c == 0:
            v = jnp.where(lane == 0, b_s, v)
        tmp_v[pl.ds(c * L, L)] = v
    src, dst = tmp_v, db_v
    s = 1
    while s < TPAD:
        for c in range(TPAD // L):
            gi = lane + c * L
            g = plsc.load_gather(src, [jnp.maximum(gi - s, 0)])
            dst[pl.ds(c * L, L)] = src[pl.ds(c * L, L)] + jnp.where(gi >= s, g, 0.0)
        src, dst = dst, src
        s *= 2

    cp.wait()

    def body(i, _):
        e = eps_v[pl.ds(i * L, L)]
        raw = jnp.minimum(jnp.maximum((e - VMIN) * INV_H, 0.0), 99.0)
        gidx = jnp.minimum(raw.astype(jnp.int32), 99)
        p0 = plsc.load_gather(pts_v, [gidx])
        p1 = plsc.load_gather(pts_v, [gidx + 1])
        idx = (gidx + jnp.where(e >= p0, 1, 0)) + jnp.where(e >= p1, 1, 0)
        sidx = jnp.maximum(idx - 1, 0)
        w = plsc.load_gather(w_v, [idx])
        sp = plsc.load_gather(pts_v, [sidx])
        db = plsc.load_gather(db_v, [sidx])
        out_v[pl.ds(i * L, L)] = (e - sp) * w + db
        return 0

    lax.fori_loop(0, CHUNK // L, body, 0)

    pltpu.sync_copy(out_v, out_hbm.at[pl.ds(base, CHUNK)])


@jax.jit
def _pwl(eps_flat, b_pad, pts_pad, p_pad):
    mesh = plsc.VectorSubcoreMesh(core_axis_name="c", subcore_axis_name="s")
    f = functools.partial(
        pl.kernel,
        mesh=mesh,
        out_type=jax.ShapeDtypeStruct((NPAD,), jnp.float32),
        compiler_params=pltpu.CompilerParams(needs_layout_passes=False),
        scratch_types=[
            pltpu.VMEM((CHUNK,), jnp.float32),
            pltpu.VMEM((CHUNK,), jnp.float32),
            pltpu.VMEM((TPAD,), jnp.float32),
            pltpu.VMEM((TPAD,), jnp.float32),
            pltpu.VMEM((TPAD,), jnp.float32),
            pltpu.VMEM((TPAD,), jnp.float32),
            pltpu.VMEM((TPAD,), jnp.float32),
            pltpu.VMEM((L,), jnp.float32),
            pltpu.SemaphoreType.DMA,
        ],
    )(_pwl_body)
    return f(eps_flat, b_pad, pts_pad, p_pad)


def kernel(eps, b, points, p):
    eps_flat = jnp.pad(eps.reshape(-1), (0, NPAD - BATCH))
    pts_pad = jnp.pad(points.reshape(-1), (0, TPAD - N_KNOTS),
                      constant_values=1e30)
    p_pad = jnp.pad(p, (0, TPAD - (N_KNOTS + 1)))
    b_pad = jnp.pad(b, (0, L - 1))
    out = _pwl(eps_flat, b_pad, pts_pad, p_pad)
    return out[:BATCH].reshape(BATCH, 1)


# speedup vs baseline: 231.2443x; 1.2758x over previous
import functools

import jax
import jax.numpy as jnp
from jax import lax
from jax.experimental import pallas as pl
from jax.experimental.pallas import tpu as pltpu
from jax.experimental.pallas import tpu_sc as plsc

VMIN = -5.0
VMAX = 5.0
N_KNOTS = 100
INT_LENGTH = (VMAX - VMIN) / (N_KNOTS - 1)
INV_H = 1.0 / INT_LENGTH
BATCH = 1000000

NC, NS, L = 2, 16, 16
NW = NC * NS
CHUNK = 31264
NPAD = NW * CHUNK
TPAD = 112


def _pwl_body(eps_hbm, b_hbm, pts_hbm, p_hbm, out_hbm,
              eps_v, out_v, pts_v, w_v, db_v, tmp_v, pp_v, b_v, sem):
    wid = lax.axis_index("s") * NC + lax.axis_index("c")

    pltpu.sync_copy(pts_hbm, pts_v)
    pltpu.sync_copy(p_hbm, pp_v)
    pltpu.sync_copy(b_hbm, b_v)
    b_s = b_v[...][0]

    base = wid * CHUNK
    cp = pltpu.async_copy(eps_hbm.at[pl.ds(base, CHUNK)], eps_v, sem)

    for c in range(TPAD // L):
        w_v[pl.ds(c * L, L)] = jnp.exp(pp_v[pl.ds(c * L, L)]) + 0.001

    lane = lax.iota(jnp.int32, L)
    for c in range(TPAD // L):
        v = INT_LENGTH * w_v[pl.ds(c * L, L)]
        if c == 0:
            v = jnp.where(lane == 0, b_s, v)
        tmp_v[pl.ds(c * L, L)] = v
    src, dst = tmp_v, db_v
    s = 1
    while s < TPAD:
        for c in range(TPAD // L):
            gi = lane + c * L
            g = plsc.load_gather(src, [jnp.maximum(gi - s, 0)])
            dst[pl.ds(c * L, L)] = src[pl.ds(c * L, L)] + jnp.where(gi >= s, g, 0.0)
        src, dst = dst, src
        s *= 2

    cp.wait()

    @plsc.parallel_loop(0, CHUNK, L, unroll=8)
    def body(i):
        e = eps_v[pl.ds(i, L)]
        raw = jnp.minimum(jnp.maximum((e - VMIN) * INV_H, 0.0), 99.0)
        gidx = jnp.minimum(raw.astype(jnp.int32), 99)
        p0 = plsc.load_gather(pts_v, [gidx])
        p1 = plsc.load_gather(pts_v, [gidx + 1])
        idx = (gidx + jnp.where(e >= p0, 1, 0)) + jnp.where(e >= p1, 1, 0)
        sidx = jnp.maximum(idx - 1, 0)
        w = plsc.load_gather(w_v, [idx])
        sp = plsc.load_gather(pts_v, [sidx])
        db = plsc.load_gather(db_v, [sidx])
        out_v[pl.ds(i, L)] = (e - sp) * w + db

    pltpu.sync_copy(out_v, out_hbm.at[pl.ds(base, CHUNK)])


@jax.jit
def _pwl(eps_flat, b_pad, pts_pad, p_pad):
    mesh = plsc.VectorSubcoreMesh(core_axis_name="c", subcore_axis_name="s")
    f = functools.partial(
        pl.kernel,
        mesh=mesh,
        out_type=jax.ShapeDtypeStruct((NPAD,), jnp.float32),
        compiler_params=pltpu.CompilerParams(needs_layout_passes=False),
        scratch_types=[
            pltpu.VMEM((CHUNK,), jnp.float32),
            pltpu.VMEM((CHUNK,), jnp.float32),
            pltpu.VMEM((TPAD,), jnp.float32),
            pltpu.VMEM((TPAD,), jnp.float32),
            pltpu.VMEM((TPAD,), jnp.float32),
            pltpu.VMEM((TPAD,), jnp.float32),
            pltpu.VMEM((TPAD,), jnp.float32),
            pltpu.VMEM((L,), jnp.float32),
            pltpu.SemaphoreType.DMA,
        ],
    )(_pwl_body)
    return f(eps_flat, b_pad, pts_pad, p_pad)


def kernel(eps, b, points, p):
    eps_flat = jnp.pad(eps.reshape(-1), (0, NPAD - BATCH))
    pts_pad = jnp.pad(points.reshape(-1), (0, TPAD - N_KNOTS),
                      constant_values=1e30)
    p_pad = jnp.pad(p, (0, TPAD - (N_KNOTS + 1)))
    b_pad = jnp.pad(b, (0, L - 1))
    out = _pwl(eps_flat, b_pad, pts_pad, p_pad)
    return out[:BATCH].reshape(BATCH, 1)


# speedup vs baseline: 231.3006x; 1.0002x over previous
import functools

import jax
import jax.numpy as jnp
from jax import lax
from jax.experimental import pallas as pl
from jax.experimental.pallas import tpu as pltpu
from jax.experimental.pallas import tpu_sc as plsc

VMIN = -5.0
VMAX = 5.0
N_KNOTS = 100
INT_LENGTH = (VMAX - VMIN) / (N_KNOTS - 1)
INV_H = 1.0 / INT_LENGTH
BATCH = 1000000

NC, NS, L = 2, 16, 16
NW = NC * NS
CHUNK = 31248
TAIL = BATCH - NW * CHUNK
TPAD = 112


def _pwl_body(eps_hbm, b_hbm, pts_hbm, p_hbm, out_hbm,
              eps_v, out_v, pts_v, w_v, db_v, tmp_v, pp_v, b_v, sem):
    wid = lax.axis_index("s") * NC + lax.axis_index("c")

    pltpu.sync_copy(pts_hbm, pts_v)
    pltpu.sync_copy(p_hbm, pp_v)
    pltpu.sync_copy(b_hbm, b_v)
    b_s = b_v[...][0]

    base = wid * CHUNK
    cp = pltpu.async_copy(eps_hbm.at[pl.ds(base, CHUNK)],
                          eps_v.at[pl.ds(0, CHUNK)], sem)

    for c in range(TPAD // L):
        w_v[pl.ds(c * L, L)] = jnp.exp(pp_v[pl.ds(c * L, L)]) + 0.001

    lane = lax.iota(jnp.int32, L)
    for c in range(TPAD // L):
        v = INT_LENGTH * w_v[pl.ds(c * L, L)]
        if c == 0:
            v = jnp.where(lane == 0, b_s, v)
        tmp_v[pl.ds(c * L, L)] = v
    src, dst = tmp_v, db_v
    s = 1
    while s < TPAD:
        for c in range(TPAD // L):
            gi = lane + c * L
            g = plsc.load_gather(src, [jnp.maximum(gi - s, 0)])
            dst[pl.ds(c * L, L)] = src[pl.ds(c * L, L)] + jnp.where(gi >= s, g, 0.0)
        src, dst = dst, src
        s *= 2

    cp.wait()

    def compute(i):
        e = eps_v[pl.ds(i, L)]
        raw = jnp.minimum(jnp.maximum((e - VMIN) * INV_H, 0.0), 99.0)
        gidx = jnp.minimum(raw.astype(jnp.int32), 99)
        p0 = plsc.load_gather(pts_v, [gidx])
        p1 = plsc.load_gather(pts_v, [gidx + 1])
        idx = (gidx + jnp.where(e >= p0, 1, 0)) + jnp.where(e >= p1, 1, 0)
        sidx = jnp.maximum(idx - 1, 0)
        w = plsc.load_gather(w_v, [idx])
        sp = plsc.load_gather(pts_v, [sidx])
        db = plsc.load_gather(db_v, [sidx])
        out_v[pl.ds(i, L)] = (e - sp) * w + db

    plsc.parallel_loop(0, CHUNK, L, unroll=8)(compute)

    @pl.when(wid == NW - 1)
    def _tail():
        pltpu.sync_copy(eps_hbm.at[pl.ds(NW * CHUNK, TAIL)],
                        eps_v.at[pl.ds(CHUNK, TAIL)])
        for t in range(TAIL // L):
            compute(CHUNK + t * L)
        pltpu.sync_copy(out_v.at[pl.ds(CHUNK, TAIL)],
                        out_hbm.at[pl.ds(NW * CHUNK, TAIL)])

    pltpu.sync_copy(out_v.at[pl.ds(0, CHUNK)], out_hbm.at[pl.ds(base, CHUNK)])


@jax.jit
def _pwl(eps_flat, b_pad, pts_pad, p_pad):
    mesh = plsc.VectorSubcoreMesh(core_axis_name="c", subcore_axis_name="s")
    f = functools.partial(
        pl.kernel,
        mesh=mesh,
        out_type=jax.ShapeDtypeStruct((BATCH,), jnp.float32),
        compiler_params=pltpu.CompilerParams(needs_layout_passes=False),
        scratch_types=[
            pltpu.VMEM((CHUNK + TAIL,), jnp.float32),
            pltpu.VMEM((CHUNK + TAIL,), jnp.float32),
            pltpu.VMEM((TPAD,), jnp.float32),
            pltpu.VMEM((TPAD,), jnp.float32),
            pltpu.VMEM((TPAD,), jnp.float32),
            pltpu.VMEM((TPAD,), jnp.float32),
            pltpu.VMEM((TPAD,), jnp.float32),
            pltpu.VMEM((L,), jnp.float32),
            pltpu.SemaphoreType.DMA,
        ],
    )(_pwl_body)
    return f(eps_flat, b_pad, pts_pad, p_pad)


def kernel(eps, b, points, p):
    eps_flat = eps.reshape(BATCH)
    pts_pad = jnp.pad(points.reshape(-1), (0, TPAD - N_KNOTS),
                      constant_values=1e30)
    p_pad = jnp.pad(p, (0, TPAD - (N_KNOTS + 1)))
    b_pad = jnp.pad(b, (0, L - 1))
    out = _pwl(eps_flat, b_pad, pts_pad, p_pad)
    return out.reshape(BATCH, 1)


# speedup vs baseline: 242.6561x; 1.0491x over previous
import functools

import jax
import jax.numpy as jnp
from jax import lax
from jax.experimental import pallas as pl
from jax.experimental.pallas import tpu as pltpu
from jax.experimental.pallas import tpu_sc as plsc

VMIN = -5.0
VMAX = 5.0
N_KNOTS = 100
INT_LENGTH = (VMAX - VMIN) / (N_KNOTS - 1)
INV_H = 1.0 / INT_LENGTH
BATCH = 1000000

NC, NS, L = 2, 16, 16
NW = NC * NS
ROWS = BATCH // L
CHUNK_R = ROWS // NW
TAIL_R = ROWS - NW * CHUNK_R
TPAD = 112


def _pwl_body(eps_hbm, b_hbm, pts_hbm, p_hbm, out_hbm,
              eps_v, out_v, pts_v, w_v, db_v, tmp_v, pp_v, b_v, sem):
    wid = lax.axis_index("s") * NC + lax.axis_index("c")
    lane = lax.iota(jnp.int32, L)
    zero16 = lane * 0

    pltpu.sync_copy(pts_hbm, pts_v)
    pltpu.sync_copy(p_hbm, pp_v)
    pltpu.sync_copy(b_hbm, b_v)
    b_s = b_v[...][0]

    base = wid * CHUNK_R
    cp = pltpu.async_copy(eps_hbm.at[0, pl.ds(base * L, CHUNK_R * L)],
                          eps_v.at[pl.ds(0, CHUNK_R * L)], sem)

    for c in range(TPAD // L):
        w_v[pl.ds(c * L, L)] = jnp.exp(pp_v[pl.ds(c * L, L)]) + 0.001

    for c in range(TPAD // L):
        v = INT_LENGTH * w_v[pl.ds(c * L, L)]
        if c == 0:
            v = jnp.where(lane == 0, b_s, v)
        tmp_v[pl.ds(c * L, L)] = v
    src, dst = tmp_v, db_v
    s = 1
    while s < TPAD:
        for c in range(TPAD // L):
            gi = lane + c * L
            g = plsc.load_gather(src, [jnp.maximum(gi - s, 0)])
            dst[pl.ds(c * L, L)] = src[pl.ds(c * L, L)] + jnp.where(gi >= s, g, 0.0)
        src, dst = dst, src
        s *= 2

    cp.wait()

    def compute(i):
        ei = lane + i * L
        e = plsc.load_gather(eps_v, [ei])
        raw = jnp.minimum(jnp.maximum((e - VMIN) * INV_H, 0.0), 99.0)
        gidx = jnp.minimum(raw.astype(jnp.int32), 99)
        p0 = plsc.load_gather(pts_v, [gidx])
        p1 = plsc.load_gather(pts_v, [gidx + 1])
        idx = (gidx + jnp.where(e >= p0, 1, 0)) + jnp.where(e >= p1, 1, 0)
        sidx = jnp.maximum(idx - 1, 0)
        w = plsc.load_gather(w_v, [idx])
        sp = plsc.load_gather(pts_v, [sidx])
        db = plsc.load_gather(db_v, [sidx])
        plsc.store_scatter(out_v, [ei], (e - sp) * w + db)

    plsc.parallel_loop(0, CHUNK_R, 1, unroll=8)(compute)

    @pl.when(wid == NW - 1)
    def _tail():
        pltpu.sync_copy(eps_hbm.at[0, pl.ds(NW * CHUNK_R * L, TAIL_R * L)],
                        eps_v.at[pl.ds(CHUNK_R * L, TAIL_R * L)])
        for t in range(TAIL_R):
            compute(CHUNK_R + t)
        pltpu.sync_copy(out_v.at[pl.ds(CHUNK_R * L, TAIL_R * L)],
                        out_hbm.at[0, pl.ds(NW * CHUNK_R * L, TAIL_R * L)])

    pltpu.sync_copy(out_v.at[pl.ds(0, CHUNK_R * L)],
                    out_hbm.at[0, pl.ds(base * L, CHUNK_R * L)])


@jax.jit
def _pwl(eps, b_pad, pts_pad, p_pad):
    mesh = plsc.VectorSubcoreMesh(core_axis_name="c", subcore_axis_name="s")
    f = functools.partial(
        pl.kernel,
        mesh=mesh,
        out_type=jax.ShapeDtypeStruct((1, BATCH), jnp.float32),
        compiler_params=pltpu.CompilerParams(needs_layout_passes=False,
                                             use_tc_tiling_on_sc=False),
        scratch_types=[
            pltpu.VMEM(((CHUNK_R + TAIL_R) * L,), jnp.float32),
            pltpu.VMEM(((CHUNK_R + TAIL_R) * L,), jnp.float32),
            pltpu.VMEM((TPAD,), jnp.float32),
            pltpu.VMEM((TPAD,), jnp.float32),
            pltpu.VMEM((TPAD,), jnp.float32),
            pltpu.VMEM((TPAD,), jnp.float32),
            pltpu.VMEM((TPAD,), jnp.float32),
            pltpu.VMEM((L,), jnp.float32),
            pltpu.SemaphoreType.DMA,
        ],
    )(_pwl_body)
    return f(eps, b_pad, pts_pad, p_pad)


def kernel(eps, b, points, p):
    pts_pad = jnp.pad(points.reshape(-1), (0, TPAD - N_KNOTS),
                      constant_values=1e30)
    p_pad = jnp.pad(p, (0, TPAD - (N_KNOTS + 1)))
    b_pad = jnp.pad(b, (0, L - 1))
    return _pwl(eps.T, b_pad, pts_pad, p_pad).T
